# baseline (speedup 1.0000x reference)
import functools

import jax
import jax.numpy as jnp
from jax import lax
from jax.experimental import pallas as pl
from jax.experimental.pallas import tpu as pltpu
from jax.experimental.pallas import tpu_sc as plsc

_VOCAB = 1000000
_EMB = 64
_HID = 64
_B = 4096
_L = 50
_N = _B * _L

_LANES = 16
_IW = 128
_XW = 128


_HVOCAB = _VOCAB // 2


def _make_sc_gather(half):
  info = plsc.get_sparse_core_info()
  nw = info.num_cores * info.num_subcores

  mesh = plsc.VectorSubcoreMesh(core_axis_name="c", subcore_axis_name="s")

  @functools.partial(
      pl.kernel,
      mesh=mesh,
      out_type=jax.ShapeDtypeStruct((_L, _B, _XW), jnp.float32),
      scratch_types=[
          pltpu.VMEM((_IW, _XW), jnp.int32),
          pltpu.VMEM((_L, _IW), jnp.int32),
          pltpu.VMEM((_L, _IW), jnp.int32),
          pltpu.VMEM((2, _IW, _EMB), jnp.float32),
          pltpu.SemaphoreType.DMA,
          pltpu.SemaphoreType.DMA,
      ],
      compiler_params=pltpu.CompilerParams(
          use_tc_tiling_on_sc=False, needs_layout_passes=False),
  )
  def gather_k(src_hbm, table_hbm, out_hbm, srcbuf_v, idx_v, bad_v, buf_v,
               sem_a, sem_b):
    wid = lax.axis_index("s") * info.num_cores + lax.axis_index("c")
    b0 = wid * _IW
    gpr = _IW // _LANES

    pltpu.sync_copy(src_hbm.at[pl.ds(b0, _IW)], srcbuf_v)

    def trans_row(t, carry):
      def trans_group(g, carry2):
        rows = g * _LANES + lax.iota(jnp.int32, _LANES)
        cols = jnp.full((_LANES,), t, jnp.int32)
        idx16 = plsc.load_gather(srcbuf_v, [rows, cols])
        loc = idx16 - half * _HVOCAB
        bad = (loc < 0) | (loc >= _HVOCAB) | (idx16 == 0)
        idx_v[t, pl.ds(g * _LANES, _LANES)] = jnp.where(bad, 0, loc)
        bad_v[t, pl.ds(g * _LANES, _LANES)] = bad.astype(jnp.int32)
        return carry2
      return lax.fori_loop(0, gpr, trans_group, carry)
    lax.fori_loop(0, _L, trans_row, 0)

    sems = (sem_a, sem_b)

    def start_gather(t, sl):
      pltpu.make_async_copy(
          table_hbm.at[idx_v.at[t]], buf_v.at[sl], sems[sl]).start()

    def wait_gather(t, sl):
      pltpu.make_async_copy(
          table_hbm.at[idx_v.at[t]], buf_v.at[sl], sems[sl]).wait()

    def zero_pad_rows(t, sl):
      def zero_group(g, carry2):
        m = bad_v[t, pl.ds(g * _LANES, _LANES)] != 0
        rowids = g * _LANES + lax.iota(jnp.int32, _LANES)
        zeros16 = jnp.zeros((_LANES,), jnp.float32)
        for s in range(_EMB):
          plsc.store_scatter(
              buf_v.at[sl], [rowids, jnp.full((_LANES,), s, jnp.int32)],
              zeros16, mask=m)
        return carry2
      lax.fori_loop(0, gpr, zero_group, 0)

    start_gather(0, 0)

    def chunk_pair(p, carry):
      for sl in range(2):
        t = p * 2 + sl

        @pl.when(t + 1 < _L)
        def _():
          start_gather(t + 1, 1 - sl)
        wait_gather(t, sl)
        zero_pad_rows(t, sl)
        pltpu.sync_copy(
            buf_v.at[sl], out_hbm.at[t, pl.ds(b0, _IW), pl.ds(0, _EMB)])
      return carry

    lax.fori_loop(0, _L // 2, chunk_pair, 0)

  return gather_k


def _gru_body(xa_ref, xb_ref, wx_ref, wh_ref, brz_ref, bin_ref, bhn_ref,
              h_ref):
  t = pl.program_id(0)

  @pl.when(t == 0)
  def _():
    h_ref[...] = jnp.zeros_like(h_ref)

  xp = xa_ref[0] + xb_ref[0]
  lane = lax.broadcasted_iota(jnp.int32, (_B, _XW), 1)
  x = jnp.where(lane < _EMB, xp, 0.0)
  h = h_ref[...]

  def mm(a, w_ref):
    return lax.dot_general(
        a, w_ref[...], (((1,), (0,)), ((), ())),
        preferred_element_type=jnp.float32)

  gi = mm(x, wx_ref)
  gh = mm(h, wh_ref)
  rz = jax.nn.sigmoid(gi[:, :2 * _HID] + gh[:, :2 * _HID] + brz_ref[...])
  r = rz[:, :_HID]
  z = rz[:, _HID:]
  gin = gi[:, 2 * _HID:] + bin_ref[...]
  ghn = gh[:, 2 * _HID:] + bhn_ref[...]
  n = jnp.tanh(gin + r * ghn)
  h_ref[...] = (1.0 - z) * n + z * h


def _run_gru(xs_a, xs_b, W_ih, W_hh, b_ih, b_hh):
  wih_t = W_ih.T
  wih_p = jnp.zeros((_XW, 3 * _HID), jnp.float32).at[:_EMB].set(wih_t)
  whh_t = W_hh.T
  brz = (b_ih[:2 * _HID] + b_hh[:2 * _HID]).reshape(1, 2 * _HID)
  bin_ = b_ih[2 * _HID:].reshape(1, _HID)
  bhn = b_hh[2 * _HID:].reshape(1, _HID)

  full = lambda shape: pl.BlockSpec(shape, lambda t: (0,) * len(shape))
  grid_spec = pltpu.PrefetchScalarGridSpec(
      num_scalar_prefetch=0,
      grid=(_L,),
      in_specs=[
          pl.BlockSpec((1, _B, _XW), lambda t: (t, 0, 0)),
          pl.BlockSpec((1, _B, _XW), lambda t: (t, 0, 0)),
          full((_XW, 3 * _HID)), full((_HID, 3 * _HID)),
          full((1, 2 * _HID)), full((1, _HID)), full((1, _HID)),
      ],
      out_specs=pl.BlockSpec((_B, _HID), lambda t: (0, 0)),
  )
  h = pl.pallas_call(
      _gru_body,
      grid_spec=grid_spec,
      out_shape=jax.ShapeDtypeStruct((_B, _HID), jnp.float32),
  )(xs_a, xs_b, wih_p, whh_t, brz, bin_, bhn)
  return h


@jax.jit
def kernel(src, emb_table, W_ih, W_hh, b_ih, b_hh):
  gather_a = _make_sc_gather(0)
  gather_b = _make_sc_gather(1)
  src_p = jnp.pad(src, ((0, 0), (0, _XW - _L)))
  table_a = lax.slice(emb_table, (0, 0), (_HVOCAB, _EMB))
  table_b = lax.slice(emb_table, (_HVOCAB, 0), (_VOCAB, _EMB))
  xs_a = gather_a(src_p, table_a)
  xs_b = gather_b(src_p, table_b)
  h = _run_gru(xs_a, xs_b, W_ih, W_hh, b_ih, b_hh)
  return h[None, :, :]


# speedup vs baseline: 6.6136x; 6.6136x over previous
import functools

import jax
import jax.numpy as jnp
from jax import lax
from jax.experimental import pallas as pl
from jax.experimental.pallas import tpu as pltpu
from jax.experimental.pallas import tpu_sc as plsc

_VOCAB = 1000000
_EMB = 64
_HID = 64
_B = 4096
_L = 50
_N = _B * _L

_LANES = 16
_IW = 128
_XW = 128


def _make_sc_gather():
  info = plsc.get_sparse_core_info()
  nw = info.num_cores * info.num_subcores

  mesh = plsc.VectorSubcoreMesh(core_axis_name="c", subcore_axis_name="s")

  @functools.partial(
      pl.kernel,
      mesh=mesh,
      out_type=jax.ShapeDtypeStruct((_L, _B, _XW), jnp.float32),
      scratch_types=[
          pltpu.VMEM((_IW, _XW), jnp.int32),
          pltpu.VMEM((_L, _IW), jnp.int32),
          pltpu.VMEM((2, _IW, _EMB), jnp.float32),
          pltpu.SemaphoreType.DMA,
          pltpu.SemaphoreType.DMA,
      ],
      compiler_params=pltpu.CompilerParams(
          use_tc_tiling_on_sc=False, needs_layout_passes=False),
  )
  def gather_k(src_hbm, table_hbm, out_hbm, srcbuf_v, idx_v, buf_v,
               sem_a, sem_b):
    wid = lax.axis_index("s") * info.num_cores + lax.axis_index("c")
    b0 = wid * _IW
    gpr = _IW // _LANES

    pltpu.sync_copy(src_hbm.at[pl.ds(b0, _IW)], srcbuf_v)

    def trans_row(t, carry):
      def trans_group(g, carry2):
        rows = g * _LANES + lax.iota(jnp.int32, _LANES)
        cols = jnp.full((_LANES,), t, jnp.int32)
        idx_v[t, pl.ds(g * _LANES, _LANES)] = plsc.load_gather(
            srcbuf_v, [rows, cols])
        return carry2
      return lax.fori_loop(0, gpr, trans_group, carry)
    lax.fori_loop(0, _L, trans_row, 0)

    sems = (sem_a, sem_b)

    def start_gather(t, sl):
      pltpu.make_async_copy(
          table_hbm.at[idx_v.at[t]], buf_v.at[sl], sems[sl]).start()

    def wait_gather(t, sl):
      pltpu.make_async_copy(
          table_hbm.at[idx_v.at[t]], buf_v.at[sl], sems[sl]).wait()

    def zero_pad_rows(t, sl):
      def cnt_group(g, acc):
        idx16 = idx_v[t, pl.ds(g * _LANES, _LANES)]
        return acc + plsc.all_reduce_population_count(idx16 == 0)
      cnt_vec = lax.fori_loop(
          0, gpr, cnt_group, jnp.zeros((_LANES,), jnp.int32))
      cnt = jnp.sum(cnt_vec)

      @pl.when(cnt > 0)
      def _():
        def zero_group(g, carry2):
          idx16 = idx_v[t, pl.ds(g * _LANES, _LANES)]
          m = idx16 == 0
          rowids = g * _LANES + lax.iota(jnp.int32, _LANES)
          zeros16 = jnp.zeros((_LANES,), jnp.float32)
          for s in range(_EMB):
            plsc.store_scatter(
                buf_v.at[sl], [rowids, jnp.full((_LANES,), s, jnp.int32)],
                zeros16, mask=m)
          return carry2
        lax.fori_loop(0, gpr, zero_group, 0)

    start_gather(0, 0)

    def chunk_pair(p, carry):
      for sl in range(2):
        t = p * 2 + sl

        @pl.when(t + 1 < _L)
        def _():
          start_gather(t + 1, 1 - sl)
        wait_gather(t, sl)
        zero_pad_rows(t, sl)
        pltpu.sync_copy(
            buf_v.at[sl], out_hbm.at[t, pl.ds(b0, _IW), pl.ds(0, _EMB)])
      return carry

    lax.fori_loop(0, _L // 2, chunk_pair, 0)

  return gather_k


def _gru_body(x_ref, wx_ref, wh_ref, brz_ref, bin_ref, bhn_ref, h_ref):
  t = pl.program_id(0)

  @pl.when(t == 0)
  def _():
    h_ref[...] = jnp.zeros_like(h_ref)

  xp = x_ref[0]
  lane = lax.broadcasted_iota(jnp.int32, (_B, _XW), 1)
  x = jnp.where(lane < _EMB, xp, 0.0)
  h = h_ref[...]

  def mm(a, w_ref):
    return lax.dot_general(
        a, w_ref[...], (((1,), (0,)), ((), ())),
        preferred_element_type=jnp.float32)

  gi = mm(x, wx_ref)
  gh = mm(h, wh_ref)
  rz = jax.nn.sigmoid(gi[:, :2 * _HID] + gh[:, :2 * _HID] + brz_ref[...])
  r = rz[:, :_HID]
  z = rz[:, _HID:]
  gin = gi[:, 2 * _HID:] + bin_ref[...]
  ghn = gh[:, 2 * _HID:] + bhn_ref[...]
  n = jnp.tanh(gin + r * ghn)
  h_ref[...] = (1.0 - z) * n + z * h


def _run_gru(xs, W_ih, W_hh, b_ih, b_hh):
  wih_t = W_ih.T
  wih_p = jnp.zeros((_XW, 3 * _HID), jnp.float32).at[:_EMB].set(wih_t)
  whh_t = W_hh.T
  brz = (b_ih[:2 * _HID] + b_hh[:2 * _HID]).reshape(1, 2 * _HID)
  bin_ = b_ih[2 * _HID:].reshape(1, _HID)
  bhn = b_hh[2 * _HID:].reshape(1, _HID)

  full = lambda shape: pl.BlockSpec(shape, lambda t: (0,) * len(shape))
  grid_spec = pltpu.PrefetchScalarGridSpec(
      num_scalar_prefetch=0,
      grid=(_L,),
      in_specs=[
          pl.BlockSpec((1, _B, _XW), lambda t: (t, 0, 0)),
          full((_XW, 3 * _HID)), full((_HID, 3 * _HID)),
          full((1, 2 * _HID)), full((1, _HID)), full((1, _HID)),
      ],
      out_specs=pl.BlockSpec((_B, _HID), lambda t: (0, 0)),
  )
  h = pl.pallas_call(
      _gru_body,
      grid_spec=grid_spec,
      out_shape=jax.ShapeDtypeStruct((_B, _HID), jnp.float32),
  )(xs, wih_p, whh_t, brz, bin_, bhn)
  return h


@jax.jit
def kernel(src, emb_table, W_ih, W_hh, b_ih, b_hh):
  gather_k = _make_sc_gather()
  src_p = jnp.pad(src, ((0, 0), (0, _XW - _L)))
  xs = gather_k(src_p, emb_table)
  h = _run_gru(xs, W_ih, W_hh, b_ih, b_hh)
  return h[None, :, :]


# speedup vs baseline: 6.7283x; 1.0173x over previous
import functools

import jax
import jax.numpy as jnp
from jax import lax
from jax.experimental import pallas as pl
from jax.experimental.pallas import tpu as pltpu
from jax.experimental.pallas import tpu_sc as plsc

_VOCAB = 1000000
_EMB = 64
_HID = 64
_B = 4096
_L = 50
_N = _B * _L

_LANES = 16
_IW = 128
_XW = 128


def _make_sc_gather():
  info = plsc.get_sparse_core_info()
  nw = info.num_cores * info.num_subcores

  mesh = plsc.VectorSubcoreMesh(core_axis_name="c", subcore_axis_name="s")

  @functools.partial(
      pl.kernel,
      mesh=mesh,
      out_type=jax.ShapeDtypeStruct((_L // 2, _B, _XW), jnp.float32),
      scratch_types=[
          pltpu.VMEM((_IW, _XW), jnp.int32),
          pltpu.VMEM((_L, _IW), jnp.int32),
          pltpu.VMEM((2, _IW, _EMB), jnp.float32),
          pltpu.SemaphoreType.DMA,
          pltpu.SemaphoreType.DMA,
      ],
      compiler_params=pltpu.CompilerParams(
          use_tc_tiling_on_sc=False, needs_layout_passes=False),
  )
  def gather_k(src_hbm, table_hbm, out_hbm, srcbuf_v, idx_v, buf_v,
               sem_a, sem_b):
    wid = lax.axis_index("s") * info.num_cores + lax.axis_index("c")
    b0 = wid * _IW
    gpr = _IW // _LANES

    pltpu.sync_copy(src_hbm.at[pl.ds(b0, _IW)], srcbuf_v)

    def trans_row(t, carry):
      def trans_group(g, carry2):
        rows = g * _LANES + lax.iota(jnp.int32, _LANES)
        cols = jnp.full((_LANES,), t, jnp.int32)
        idx_v[t, pl.ds(g * _LANES, _LANES)] = plsc.load_gather(
            srcbuf_v, [rows, cols])
        return carry2
      return lax.fori_loop(0, gpr, trans_group, carry)
    lax.fori_loop(0, _L, trans_row, 0)

    sems = (sem_a, sem_b)

    def start_gather(t, sl):
      pltpu.make_async_copy(
          table_hbm.at[idx_v.at[t]], buf_v.at[sl], sems[sl]).start()

    def wait_gather(t, sl):
      pltpu.make_async_copy(
          table_hbm.at[idx_v.at[t]], buf_v.at[sl], sems[sl]).wait()

    def zero_pad_rows(t, sl):
      def cnt_group(g, acc):
        idx16 = idx_v[t, pl.ds(g * _LANES, _LANES)]
        return acc + plsc.all_reduce_population_count(idx16 == 0)
      cnt_vec = lax.fori_loop(
          0, gpr, cnt_group, jnp.zeros((_LANES,), jnp.int32))
      cnt = jnp.sum(cnt_vec)

      @pl.when(cnt > 0)
      def _():
        def zero_group(g, carry2):
          idx16 = idx_v[t, pl.ds(g * _LANES, _LANES)]
          m = idx16 == 0
          rowids = g * _LANES + lax.iota(jnp.int32, _LANES)
          zeros16 = jnp.zeros((_LANES,), jnp.float32)
          for s in range(_EMB):
            plsc.store_scatter(
                buf_v.at[sl], [rowids, jnp.full((_LANES,), s, jnp.int32)],
                zeros16, mask=m)
          return carry2
        lax.fori_loop(0, gpr, zero_group, 0)

    start_gather(0, 0)

    def chunk_pair(p, carry):
      for sl in range(2):
        t = p * 2 + sl

        @pl.when(t + 1 < _L)
        def _():
          start_gather(t + 1, 1 - sl)
        wait_gather(t, sl)
        zero_pad_rows(t, sl)
        pltpu.sync_copy(
            buf_v.at[sl],
            out_hbm.at[p, pl.ds(b0, _IW), pl.ds(sl * _EMB, _EMB)])
      return carry

    lax.fori_loop(0, _L // 2, chunk_pair, 0)

  return gather_k


def _gru_body(x_ref, wxe_ref, wxo_ref, wh_ref, brz_ref, bin_ref, bhn_ref,
              h_ref):
  t = pl.program_id(0)

  @pl.when(t == 0)
  def _():
    h_ref[...] = jnp.zeros_like(h_ref)

  xp = x_ref[0]
  h = h_ref[...]

  def mm(a, w):
    return lax.dot_general(
        a, w, (((1,), (0,)), ((), ())),
        preferred_element_type=jnp.float32)

  def step(h, gi):
    gh = mm(h, wh_ref[...])
    rz = jax.nn.sigmoid(gi[:, :2 * _HID] + gh[:, :2 * _HID] + brz_ref[...])
    r = rz[:, :_HID]
    z = rz[:, _HID:]
    gin = gi[:, 2 * _HID:] + bin_ref[...]
    ghn = gh[:, 2 * _HID:] + bhn_ref[...]
    n = jnp.tanh(gin + r * ghn)
    return (1.0 - z) * n + z * h

  h = step(h, mm(xp, wxe_ref[...]))
  h = step(h, mm(xp, wxo_ref[...]))
  h_ref[...] = h


def _run_gru(xs, W_ih, W_hh, b_ih, b_hh):
  wih_t = W_ih.T
  wxe = jnp.zeros((_XW, 3 * _HID), jnp.float32).at[:_EMB].set(wih_t)
  wxo = jnp.zeros((_XW, 3 * _HID), jnp.float32).at[_EMB:].set(wih_t)
  whh_t = W_hh.T
  brz = (b_ih[:2 * _HID] + b_hh[:2 * _HID]).reshape(1, 2 * _HID)
  bin_ = b_ih[2 * _HID:].reshape(1, _HID)
  bhn = b_hh[2 * _HID:].reshape(1, _HID)

  full = lambda shape: pl.BlockSpec(shape, lambda t: (0,) * len(shape))
  grid_spec = pltpu.PrefetchScalarGridSpec(
      num_scalar_prefetch=0,
      grid=(_L // 2,),
      in_specs=[
          pl.BlockSpec((1, _B, _XW), lambda t: (t, 0, 0)),
          full((_XW, 3 * _HID)), full((_XW, 3 * _HID)),
          full((_HID, 3 * _HID)),
          full((1, 2 * _HID)), full((1, _HID)), full((1, _HID)),
      ],
      out_specs=pl.BlockSpec((_B, _HID), lambda t: (0, 0)),
  )
  h = pl.pallas_call(
      _gru_body,
      grid_spec=grid_spec,
      out_shape=jax.ShapeDtypeStruct((_B, _HID), jnp.float32),
  )(xs, wxe, wxo, whh_t, brz, bin_, bhn)
  return h


@jax.jit
def kernel(src, emb_table, W_ih, W_hh, b_ih, b_hh):
  gather_k = _make_sc_gather()
  src_p = jnp.pad(src, ((0, 0), (0, _XW - _L)))
  xs = gather_k(src_p, emb_table)
  h = _run_gru(xs, W_ih, W_hh, b_ih, b_hh)
  return h[None, :, :]
